# baseline (speedup 1.0000x reference)
import functools

import jax
import jax.numpy as jnp
from jax import lax
from jax.experimental import pallas as pl
from jax.experimental.pallas import tpu as pltpu
from jax.experimental.pallas import tpu_sc as plsc

_NUM_USERS = 1000000
_NUM_ITEMS = 1000
_DIM = 32
_BATCH = 16384

_NBAND = 128 // _DIM
_RB = 8192
_RBH = _RB // 2
_UB = _RB * _NBAND
_GRID_R = -(-_NUM_USERS // _UB)
_NLINES = _GRID_R * _RBH

_NC = 2
_NS = 16
_NW = _NC * _NS
_BPW = _BATCH // _NW
_CHUNK = 128
_NCHUNK = _BPW // _CHUNK
_L = 16

_BM = 2048


def _relayout_body(x_ref, out_ref):
    x = x_ref[...]
    z = jnp.concatenate(
        [x[:, s * _RB:(s + 1) * _RB] for s in range(_NBAND)], axis=0
    )
    zi = lax.bitcast_convert_type(z, jnp.int32)
    r = zi + jnp.int32(0x7FFF) + ((zi >> jnp.int32(16)) & jnp.int32(1))
    lo = r[:, :_RBH]
    hi = r[:, _RBH:]
    packed = ((lo >> jnp.int32(16)) & jnp.int32(0xFFFF)) | (
        hi & jnp.int32(-65536)
    )
    out_ref[...] = packed.T


@functools.cache
def _build_relayout():
    return pl.pallas_call(
        _relayout_body,
        grid=(_GRID_R,),
        in_specs=[pl.BlockSpec((_DIM, _UB), lambda i: (0, i))],
        out_specs=pl.BlockSpec((_RBH, 128), lambda i: (i, 0)),
        out_shape=jax.ShapeDtypeStruct((_NLINES, 128), jnp.int32),
    )


@functools.cache
def _build_sc_gather():
    mesh = plsc.VectorSubcoreMesh(
        core_axis_name="c", subcore_axis_name="s",
        num_cores=_NC, num_subcores=_NS,
    )

    @functools.partial(
        pl.kernel,
        mesh=mesh,
        out_type=jax.ShapeDtypeStruct((_BATCH, 128), jnp.int32),
        scratch_types=[
            pltpu.VMEM((_BPW,), jnp.int32),
            pltpu.VMEM((_BPW,), jnp.int32),
            pltpu.VMEM((_BPW, 128), jnp.int32),
            pltpu.SemaphoreType.DMA,
        ],
    )
    def sc_gather(table_hbm, idx_hbm, out_hbm, idx_v, line_v, rows_v, sem):
        wid = lax.axis_index("s") * _NC + lax.axis_index("c")
        base = wid * _BPW
        pltpu.sync_copy(idx_hbm.at[pl.ds(base, _BPW)], idx_v)
        for j in range(_BPW // _L):
            u = idx_v[pl.ds(j * _L, _L)]
            line_v[pl.ds(j * _L, _L)] = (
                ((u >> jnp.int32(15)) << jnp.int32(12))
                + (u & jnp.int32(_RBH - 1))
            )
        copies = []
        for j in range(_NCHUNK):
            copies.append(
                pltpu.async_copy(
                    table_hbm.at[line_v.at[pl.ds(j * _CHUNK, _CHUNK)]],
                    rows_v.at[pl.ds(j * _CHUNK, _CHUNK)],
                    sem,
                )
            )
        for c in copies:
            c.wait()
        pltpu.sync_copy(rows_v, out_hbm.at[pl.ds(base, _BPW)])

    return sc_gather


def _logits_body(g_ref, idx_ref, alpha_t_ref, out_ref):
    gi = g_ref[...]
    lo = lax.bitcast_convert_type(
        gi << jnp.int32(16), jnp.float32
    )
    hi = lax.bitcast_convert_type(
        gi & jnp.int32(-65536), jnp.float32
    )
    selr = idx_ref[...].reshape(1, _BM)
    sel = lax.transpose(selr, (1, 0))
    band = (sel >> jnp.int32(13)) & jnp.int32(_NBAND - 1)
    odd = ((sel >> jnp.int32(12)) & jnp.int32(1)) == 1
    theta = jnp.zeros((gi.shape[0], _DIM), jnp.float32)
    for s in range(_NBAND):
        piece = jnp.where(
            odd, hi[:, s * _DIM:(s + 1) * _DIM],
            lo[:, s * _DIM:(s + 1) * _DIM],
        )
        theta = theta + jnp.where(band == s, piece, 0.0)
    alpha_t = alpha_t_ref[...]
    ut = lax.dot_general(
        alpha_t, theta, (((0,), (1,)), ((), ())),
        preferred_element_type=jnp.float32,
    )
    e = jnp.exp(ut)
    ssum = jnp.sum(e, axis=0, keepdims=True)
    out_ref[...] = ut - jnp.log(ssum)


@functools.cache
def _build_logits():
    return pl.pallas_call(
        _logits_body,
        grid=(_BATCH // _BM,),
        in_specs=[
            pl.BlockSpec((_BM, 128), lambda i: (i, 0)),
            pl.BlockSpec((1, 1, _BM), lambda i: (i, 0, 0)),
            pl.BlockSpec((_DIM, _NUM_ITEMS), lambda i: (0, 0)),
        ],
        out_specs=pl.BlockSpec((_NUM_ITEMS, _BM), lambda i: (0, i)),
        out_shape=jax.ShapeDtypeStruct((_NUM_ITEMS, _BATCH), jnp.float32),
    )


def kernel(user_index, theta_user, alpha_item):
    tt = theta_user.T
    table = _build_relayout()(tt)
    lines = _build_sc_gather()(table, user_index)
    idx2d = user_index.reshape(_BATCH // _BM, 1, _BM)
    out_t = _build_logits()(lines, idx2d, alpha_item.T)
    return out_t.T


# speedup vs baseline: 1.0201x; 1.0201x over previous
import functools

import jax
import jax.numpy as jnp
from jax import lax
from jax.experimental import pallas as pl
from jax.experimental.pallas import tpu as pltpu
from jax.experimental.pallas import tpu_sc as plsc

_NUM_USERS = 1000000
_NUM_ITEMS = 1000
_DIM = 32
_BATCH = 16384

_NBAND = 128 // _DIM
_RB = 16384
_RBH = _RB // 2
_UB = _RB * _NBAND
_GRID_R = -(-_NUM_USERS // _UB)
_NLINES = _GRID_R * _RBH

_NC = 2
_NS = 16
_NW = _NC * _NS
_BPW = _BATCH // _NW
_CHUNK = 128
_NCHUNK = _BPW // _CHUNK
_L = 16

_BM = 2048


def _relayout_body(x_ref, out_ref):
    x = x_ref[...]
    z = jnp.concatenate(
        [x[:, s * _RB:(s + 1) * _RB] for s in range(_NBAND)], axis=0
    )
    zi = lax.bitcast_convert_type(z, jnp.int32)
    r = zi + jnp.int32(0x7FFF) + ((zi >> jnp.int32(16)) & jnp.int32(1))
    lo = r[:, :_RBH]
    hi = r[:, _RBH:]
    packed = ((lo >> jnp.int32(16)) & jnp.int32(0xFFFF)) | (
        hi & jnp.int32(-65536)
    )
    out_ref[...] = packed.T


@functools.cache
def _build_relayout():
    return pl.pallas_call(
        _relayout_body,
        grid=(_GRID_R,),
        in_specs=[pl.BlockSpec((_DIM, _UB), lambda i: (0, i))],
        out_specs=pl.BlockSpec((_RBH, 128), lambda i: (i, 0)),
        out_shape=jax.ShapeDtypeStruct((_NLINES, 128), jnp.int32),
    )


@functools.cache
def _build_sc_gather():
    mesh = plsc.VectorSubcoreMesh(
        core_axis_name="c", subcore_axis_name="s",
        num_cores=_NC, num_subcores=_NS,
    )

    @functools.partial(
        pl.kernel,
        mesh=mesh,
        out_type=jax.ShapeDtypeStruct((_BATCH, 128), jnp.int32),
        scratch_types=[
            pltpu.VMEM((_BPW,), jnp.int32),
            pltpu.VMEM((_BPW,), jnp.int32),
            pltpu.VMEM((_BPW, 128), jnp.int32),
            pltpu.SemaphoreType.DMA,
        ],
    )
    def sc_gather(table_hbm, idx_hbm, out_hbm, idx_v, line_v, rows_v, sem):
        wid = lax.axis_index("s") * _NC + lax.axis_index("c")
        base = wid * _BPW
        pltpu.sync_copy(idx_hbm.at[pl.ds(base, _BPW)], idx_v)
        for j in range(_BPW // _L):
            u = idx_v[pl.ds(j * _L, _L)]
            line_v[pl.ds(j * _L, _L)] = (
                ((u >> jnp.int32(16)) << jnp.int32(13))
                + (u & jnp.int32(_RBH - 1))
            )
        copies = []
        for j in range(_NCHUNK):
            copies.append(
                pltpu.async_copy(
                    table_hbm.at[line_v.at[pl.ds(j * _CHUNK, _CHUNK)]],
                    rows_v.at[pl.ds(j * _CHUNK, _CHUNK)],
                    sem,
                )
            )
        for c in copies:
            c.wait()
        pltpu.sync_copy(rows_v, out_hbm.at[pl.ds(base, _BPW)])

    return sc_gather


def _logits_body(g_ref, idx_ref, alpha_t_ref, out_ref):
    gi = g_ref[...]
    lo = lax.bitcast_convert_type(
        gi << jnp.int32(16), jnp.float32
    )
    hi = lax.bitcast_convert_type(
        gi & jnp.int32(-65536), jnp.float32
    )
    selr = idx_ref[...].reshape(1, _BM)
    sel = lax.transpose(selr, (1, 0))
    band = (sel >> jnp.int32(14)) & jnp.int32(_NBAND - 1)
    odd = ((sel >> jnp.int32(13)) & jnp.int32(1)) == 1
    theta = jnp.zeros((gi.shape[0], _DIM), jnp.float32)
    for s in range(_NBAND):
        piece = jnp.where(
            odd, hi[:, s * _DIM:(s + 1) * _DIM],
            lo[:, s * _DIM:(s + 1) * _DIM],
        )
        theta = theta + jnp.where(band == s, piece, 0.0)
    alpha_t = alpha_t_ref[...]
    ut = lax.dot_general(
        alpha_t, theta, (((0,), (1,)), ((), ())),
        preferred_element_type=jnp.float32,
    )
    e = jnp.exp(ut)
    ssum = jnp.sum(e, axis=0, keepdims=True)
    out_ref[...] = ut - jnp.log(ssum)


@functools.cache
def _build_logits():
    return pl.pallas_call(
        _logits_body,
        grid=(_BATCH // _BM,),
        in_specs=[
            pl.BlockSpec((_BM, 128), lambda i: (i, 0)),
            pl.BlockSpec((1, 1, _BM), lambda i: (i, 0, 0)),
            pl.BlockSpec((_DIM, _NUM_ITEMS), lambda i: (0, 0)),
        ],
        out_specs=pl.BlockSpec((_NUM_ITEMS, _BM), lambda i: (0, i)),
        out_shape=jax.ShapeDtypeStruct((_NUM_ITEMS, _BATCH), jnp.float32),
    )


def kernel(user_index, theta_user, alpha_item):
    tt = theta_user.T
    table = _build_relayout()(tt)
    lines = _build_sc_gather()(table, user_index)
    idx2d = user_index.reshape(_BATCH // _BM, 1, _BM)
    out_t = _build_logits()(lines, idx2d, alpha_item.T)
    return out_t.T
